# baseline (speedup 1.0000x reference)
import functools
import jax
import jax.numpy as jnp
from jax import lax
from jax.experimental import pallas as pl
from jax.experimental.pallas import tpu as pltpu
from jax.experimental.pallas import tpu_sc as plsc

_NUM_BINS = 100000
_SALT_ADD = (42 * 0x9E3779B9) & 0xFFFFFFFF

_NC = 2
_NS = 16
_NW = _NC * _NS
_LANES = 16


def _hash_vec(z):
    z = z + jnp.uint32(_SALT_ADD)
    z = (z ^ (z >> 16)) * jnp.uint32(0x85EBCA6B)
    z = (z ^ (z >> 13)) * jnp.uint32(0xC2B2AE35)
    z = z ^ (z >> 16)
    q = z // jnp.uint32(_NUM_BINS)
    return z - q * jnp.uint32(_NUM_BINS)


def _make_sc_kernel(m, n, chunk):
    per_w = n // _NW
    n_chunks = per_w // chunk
    vec_per_row = chunk // _LANES
    mesh = plsc.VectorSubcoreMesh(core_axis_name="c", subcore_axis_name="s")

    @functools.partial(
        pl.kernel,
        mesh=mesh,
        out_type=jax.ShapeDtypeStruct((m, n), jnp.int32),
        compiler_params=pltpu.CompilerParams(use_tc_tiling_on_sc=True),
        scratch_types=[
            pltpu.VMEM((2, m, chunk), jnp.int32),
            pltpu.VMEM((2, m, chunk), jnp.int32),
            pltpu.SemaphoreType.DMA,
            pltpu.SemaphoreType.DMA,
            pltpu.SemaphoreType.DMA,
            pltpu.SemaphoreType.DMA,
        ],
    )
    def sc_hash(x_hbm, out_hbm, inbuf, outbuf, insem0, insem1, outsem0, outsem1):
        wid = lax.axis_index("s") * _NC + lax.axis_index("c")
        base = wid * per_w
        insems = (insem0, insem1)
        outsems = (outsem0, outsem1)

        def in_copy(g):
            slot = g % 2
            return pltpu.make_async_copy(
                x_hbm.at[:, pl.ds(base + g * chunk, chunk)], inbuf.at[slot],
                insems[slot])

        def out_copy(g):
            slot = g % 2
            return pltpu.make_async_copy(
                outbuf.at[slot], out_hbm.at[:, pl.ds(base + g * chunk, chunk)],
                outsems[slot])

        in_copy(0).start()
        for g in range(n_chunks):
            slot = g % 2
            in_copy(g).wait()
            if g + 1 < n_chunks:
                in_copy(g + 1).start()
            if g >= 2:
                out_copy(g - 2).wait()

            def row_body(r, c, slot=slot):
                for k in range(vec_per_row):
                    v = inbuf[slot, r, pl.ds(k * _LANES, _LANES)]
                    h = _hash_vec(v.astype(jnp.uint32))
                    outbuf[slot, r, pl.ds(k * _LANES, _LANES)] = h.astype(jnp.int32)
                return c

            lax.fori_loop(0, m, row_body, 0)
            out_copy(g).start()
        out_copy(n_chunks - 2).wait()
        out_copy(n_chunks - 1).wait()

    return sc_hash


def kernel(inputs):
    n, m = inputs.shape
    xt = jnp.swapaxes(inputs, 0, 1)
    out_t = _make_sc_kernel(m, n, 128)(xt)
    return jnp.swapaxes(out_t, 0, 1)


# speedup vs baseline: 3.0351x; 3.0351x over previous
import jax
import jax.numpy as jnp
from jax.experimental import pallas as pl
from jax.experimental.pallas import tpu as pltpu

_NUM_BINS = 100000
_SALT_ADD = (42 * 0x9E3779B9) & 0xFFFFFFFF


def _hash_block(x_ref, o_ref):
    z = x_ref[...].astype(jnp.uint32)
    z = z + jnp.uint32(_SALT_ADD)
    z = (z ^ (z >> 16)) * jnp.uint32(0x85EBCA6B)
    z = (z ^ (z >> 13)) * jnp.uint32(0xC2B2AE35)
    z = z ^ (z >> 16)
    q = z // jnp.uint32(_NUM_BINS)
    r = z - q * jnp.uint32(_NUM_BINS)
    o_ref[...] = r.astype(jnp.int32)


def kernel(inputs):
    n, m = inputs.shape
    xt = jnp.swapaxes(inputs, 0, 1)
    xt = pltpu.with_memory_space_constraint(xt, pltpu.MemorySpace.HBM)
    grid = 5
    br = m // grid
    out_t = pl.pallas_call(
        _hash_block,
        grid=(grid,),
        in_specs=[pl.BlockSpec((br, n), lambda i: (i, 0))],
        out_specs=pl.BlockSpec((br, n), lambda i: (i, 0)),
        out_shape=jax.ShapeDtypeStruct((m, n), jnp.int32),
    )(xt)
    return jnp.swapaxes(out_t, 0, 1)
